# baseline (speedup 1.0000x reference)
import functools

import jax
import jax.numpy as jnp
from jax import lax
from jax.experimental import pallas as pl
from jax.experimental.pallas import tpu as pltpu
from jax.experimental.pallas import tpu_sc as plsc

D_MODEL = 64
SEQ = 50
LANES = 16
NW = 32

SEQ_PER_CHUNK = 2
CHUNK = SEQ_PER_CHUNK * SEQ
CHUNK_PAD = 104


def _pos_encoding(num_words, d_model):
    pos = jnp.arange(num_words, dtype=jnp.float32)[:, None]
    i = jnp.arange(d_model, dtype=jnp.float32)[None, :]
    denom = jnp.power(10000.0, 2.0 * i / d_model)
    angle = pos / denom
    even_mask = (jnp.arange(d_model) % 2 == 0)[None, :]
    return jnp.where(even_mask, jnp.sin(angle), jnp.cos(angle))


def _sc_embed(x_flat, table, pe, n_chunks, chunks_per_w):
    mesh = plsc.VectorSubcoreMesh(core_axis_name="c", subcore_axis_name="s")
    nc = mesh.num_cores

    @functools.partial(
        pl.kernel,
        out_type=jax.ShapeDtypeStruct((n_chunks * CHUNK_PAD, D_MODEL), jnp.float32),
        mesh=mesh,
        scratch_types=[
            pltpu.VMEM((CHUNK_PAD,), jnp.int32),
            pltpu.VMEM((CHUNK_PAD,), jnp.int32),
            pltpu.VMEM((CHUNK_PAD, D_MODEL), jnp.float32),
            pltpu.VMEM((CHUNK_PAD, D_MODEL), jnp.float32),
            pltpu.VMEM((SEQ, D_MODEL), jnp.float32),
            [pltpu.SemaphoreType.DMA] * 2,
            [pltpu.SemaphoreType.DMA] * 2,
        ],
        compiler_params=pltpu.CompilerParams(use_tc_tiling_on_sc=False),
    )
    def k(x_hbm, table_hbm, pe_hbm, out_hbm,
          idx0, idx1, rows0, rows1, pe_v, gsem, ssem):
        wid = lax.axis_index("s") * nc + lax.axis_index("c")
        idx = [idx0, idx1]
        rows = [rows0, rows1]
        pltpu.sync_copy(pe_hbm, pe_v)

        def chunk_of(kk):
            return kk * NW + wid

        def fire_gather(kk, p):
            pltpu.sync_copy(
                x_hbm.at[pl.ds(chunk_of(kk) * CHUNK_PAD, CHUNK_PAD)], idx[p]
            )
            pltpu.async_copy(table_hbm.at[idx[p]], rows[p], gsem[p])

        def drain_gather(p):
            pltpu.make_async_copy(
                table_hbm.at[pl.ds(0, CHUNK_PAD)], rows[p], gsem[p]
            ).wait()

        def drain_store(p):
            pltpu.make_async_copy(
                table_hbm.at[pl.ds(0, CHUNK_PAD)], rows[p], ssem[p]
            ).wait()

        def add_pe_and_store(kk, p):
            rp = rows[p]

            def pe_body(s, _):
                pe_vecs = [pe_v[s, pl.ds(q * LANES, LANES)]
                           for q in range(D_MODEL // LANES)]
                for rseq in range(SEQ_PER_CHUNK):
                    r = rseq * SEQ + s
                    for q in range(D_MODEL // LANES):
                        sl = pl.ds(q * LANES, LANES)
                        rp[r, sl] = rp[r, sl] + pe_vecs[q]
                return 0

            lax.fori_loop(0, SEQ, pe_body, 0)
            pltpu.async_copy(
                rp,
                out_hbm.at[pl.ds(chunk_of(kk) * CHUNK_PAD, CHUNK_PAD)],
                ssem[p],
            )

        fire_gather(0, 0)

        drain_gather(0)
        fire_gather(1, 1)
        add_pe_and_store(0, 0)

        @pl.loop(1, chunks_per_w - 1, step=2)
        def body(t):
            for b in range(2):
                kk = t + b
                p = (1 + b) % 2
                drain_gather(p)
                drain_store(1 - p)
                fire_gather(kk + 1, 1 - p)
                add_pe_and_store(kk, p)

        p_last = (chunks_per_w - 1) % 2
        drain_gather(p_last)
        drain_store(1 - p_last)
        add_pe_and_store(chunks_per_w - 1, p_last)
        drain_store(p_last)

    return k(x_flat, table, pe)


def kernel(x, table):
    if x.ndim == 1:
        x = x[None, :]
    batch, seq = x.shape
    d_model = table.shape[1]
    pe = _pos_encoding(seq, d_model).astype(jnp.float32)

    rows_total = batch * seq
    n_chunks = rows_total // CHUNK
    chunks_per_w = n_chunks // NW

    x_pad = jnp.pad(x.reshape(n_chunks, CHUNK),
                    ((0, 0), (0, CHUNK_PAD - CHUNK))).reshape(-1)
    out = _sc_embed(x_pad, table, pe, n_chunks, chunks_per_w)
    out = out.reshape(n_chunks, CHUNK_PAD, d_model)[:, :CHUNK, :]
    return out.reshape(batch, seq, d_model)
